# Initial kernel scaffold; baseline (speedup 1.0000x reference)
import functools

import jax
import jax.numpy as jnp
from jax import lax
from jax.experimental import pallas as pl
from jax.experimental.pallas import tpu as pltpu
from jax.experimental.pallas import tpu_sc as plsc

_N = 10000
_E = 320000
_D = 128
_G = 64
_T = 128
_L = 5

_NC = 2
_NS = 16
_NW = _NC * _NS
_K = 80
_EPW = _E // _NW
_NB = _EPW // _K
_RPS = _N // _NS
_ZR = 125
_NZ = _RPS // _ZR

_R = 1000
_NBLK = _N // _R

_sc_mesh = plsc.VectorSubcoreMesh(core_axis_name="c", subcore_axis_name="s")


@functools.partial(
    pl.kernel,
    out_type=jax.ShapeDtypeStruct((_NC, _N, _D), jnp.float32),
    mesh=_sc_mesh,
    scratch_types=[
        pltpu.VMEM((_NB, _K), jnp.int32),
        pltpu.VMEM((_NB, _K), jnp.int32),
        pltpu.VMEM((_K, _D), jnp.float32),
        pltpu.VMEM((_ZR, _D), jnp.float32),
        pltpu.VMEM_SHARED((_N, _D), jnp.float32),
        pltpu.SemaphoreType.DMA,
    ],
)
def _sc_edge_agg(h_hbm, srcb_hbm, dstb_hbm, out_hbm,
                 src_v, dst_v, rows_v, stage_v, agg_sh, sem):
    c = lax.axis_index("c")
    s = lax.axis_index("s")
    wid = c * _NS + s

    def _z(i, carry):
        stage_v[i // 8, pl.ds((i % 8) * 16, 16)] = jnp.zeros((16,), jnp.float32)
        return carry
    lax.fori_loop(0, _ZR * 8, _z, 0)

    def _zs(t, carry):
        pltpu.sync_copy(stage_v, agg_sh.at[pl.ds(s * _RPS + t * _ZR, _ZR)])
        return carry
    lax.fori_loop(0, _NZ, _zs, 0)
    plsc.subcore_barrier()

    pltpu.sync_copy(srcb_hbm.at[pl.ds(wid * _NB, _NB)], src_v)
    pltpu.sync_copy(dstb_hbm.at[pl.ds(wid * _NB, _NB)], dst_v)

    def _edge(t, carry):
        pltpu.async_copy(h_hbm.at[src_v.at[t]], rows_v, sem).wait()
        pltpu.sync_copy(rows_v, agg_sh.at[dst_v.at[t]], add=True)
        return carry
    lax.fori_loop(0, _NB, _edge, 0)
    plsc.subcore_barrier()

    def _wo(t, carry):
        r0 = s * _RPS + t * _ZR
        pltpu.sync_copy(agg_sh.at[pl.ds(r0, _ZR)], stage_v)
        pltpu.sync_copy(stage_v, out_hbm.at[c, pl.ds(r0, _ZR)])
        return carry
    lax.fori_loop(0, _NZ, _wo, 0)


def _dot(a, b, dims):
    return lax.dot_general(a, b, (dims, ((), ())),
                           precision=lax.Precision.HIGHEST,
                           preferred_element_type=jnp.float32)


def _tc_layer_body(scale_ref, h_ref, p_ref, w_ref, b_ref, o_ref):
    z = scale_ref[0, 0] * h_ref[...] + p_ref[0] + p_ref[1]
    y = _dot(z, w_ref[...], ((1,), (0,)))
    o_ref[...] = jnp.maximum(y + b_ref[...], 0.0)


_tc_layer = pl.pallas_call(
    _tc_layer_body,
    grid=(_NBLK,),
    in_specs=[
        pl.BlockSpec(memory_space=pltpu.SMEM),
        pl.BlockSpec((_R, _D), lambda i: (i, 0)),
        pl.BlockSpec((_NC, _R, _D), lambda i: (0, i, 0)),
        pl.BlockSpec((_D, _D), lambda i: (0, 0)),
        pl.BlockSpec((1, _D), lambda i: (0, 0)),
    ],
    out_specs=pl.BlockSpec((_R, _D), lambda i: (i, 0)),
    out_shape=jax.ShapeDtypeStruct((_N, _D), jnp.float32),
)


def _tc_final_body(scale_ref, h_ref, p_ref, w4_ref, b4_ref, wn_ref, bn_ref,
                   wo_ref, bo_ref, bat_ref, o_ref, sums_ref, cnt_ref):
    i = pl.program_id(0)

    @pl.when(i == 0)
    def _():
        sums_ref[...] = jnp.zeros((_G, _D), jnp.float32)
        cnt_ref[...] = jnp.zeros((_G, _D), jnp.float32)

    z = scale_ref[0, 0] * h_ref[...] + p_ref[0] + p_ref[1]
    h5 = jnp.maximum(_dot(z, w4_ref[...], ((1,), (0,))) + b4_ref[...], 0.0)
    h6 = jnp.maximum(_dot(h5, wn_ref[...], ((1,), (0,))) + bn_ref[...], 0.0)
    oh = (bat_ref[...] == lax.broadcasted_iota(jnp.float32, (_R, _G), 1))
    oh = oh.astype(jnp.float32)
    sums_ref[...] += _dot(oh, h6, ((0,), (0,)))
    cnt_ref[...] += jnp.sum(oh, axis=0)[:, None]

    @pl.when(i == _NBLK - 1)
    def _():
        hg = sums_ref[...] / jnp.maximum(cnt_ref[...], 1.0)
        o_ref[...] = _dot(hg, wo_ref[...], ((1,), (0,))) + bo_ref[...]


_tc_final = pl.pallas_call(
    _tc_final_body,
    grid=(_NBLK,),
    in_specs=[
        pl.BlockSpec(memory_space=pltpu.SMEM),
        pl.BlockSpec((_R, _D), lambda i: (i, 0)),
        pl.BlockSpec((_NC, _R, _D), lambda i: (0, i, 0)),
        pl.BlockSpec((_D, _D), lambda i: (0, 0)),
        pl.BlockSpec((1, _D), lambda i: (0, 0)),
        pl.BlockSpec((_D, _D), lambda i: (0, 0)),
        pl.BlockSpec((1, _D), lambda i: (0, 0)),
        pl.BlockSpec((_D, _T), lambda i: (0, 0)),
        pl.BlockSpec((1, _T), lambda i: (0, 0)),
        pl.BlockSpec((_R, 1), lambda i: (i, 0)),
    ],
    out_specs=pl.BlockSpec((_G, _T), lambda i: (0, 0)),
    out_shape=jax.ShapeDtypeStruct((_G, _T), jnp.float32),
    scratch_shapes=[
        pltpu.VMEM((_G, _D), jnp.float32),
        pltpu.VMEM((_G, _D), jnp.float32),
    ],
)


def kernel(x, edge_index, batch, Wl, bl, eps, Wn2n, bn2n, Wout, bout):
    src_b = edge_index[0].reshape(_NW * _NB, _K)
    dst_b = edge_index[1].reshape(_NW * _NB, _K)
    batch_f = batch.astype(jnp.float32).reshape(_N, 1)
    h = x
    for l in range(_L):
        parts = _sc_edge_agg(h, src_b, dst_b)
        scale = (1.0 + eps[l]).reshape(1, 1)
        b_l = bl[l].reshape(1, _D)
        if l < _L - 1:
            h = _tc_layer(scale, h, parts, Wl[l], b_l)
        else:
            out = _tc_final(scale, h, parts, Wl[l], b_l,
                            Wn2n, bn2n.reshape(1, _D),
                            Wout, bout.reshape(1, _T), batch_f)
    return out


# speedup vs baseline: 7.0117x; 7.0117x over previous
import functools

import jax
import jax.numpy as jnp
from jax import lax
from jax.experimental import pallas as pl
from jax.experimental.pallas import tpu as pltpu
from jax.experimental.pallas import tpu_sc as plsc

_N = 10000
_E = 320000
_D = 128
_G = 64
_T = 128
_L = 5

_NC = 2
_NS = 16
_NW = _NC * _NS
_K = 80
_EPW = _E // _NW
_NB = _EPW // _K
_NP = 10240
_RPS = _NP // _NS
_ZR = _K
_NZ = _RPS // _ZR

_R = 1000
_NBLK = _N // _R

def _sc_edge_agg_body(h_hbm, srcb_hbm, dstb_hbm, out_hbm,
                      src_v, dst_v, rows_v, agg_sh, sem):
    c = lax.axis_index("c")
    s = lax.axis_index("s")
    wid = c * _NS + s

    def _z(i, carry):
        rows_v[i // 8, pl.ds((i % 8) * 16, 16)] = jnp.zeros((16,), jnp.float32)
        return carry
    lax.fori_loop(0, _ZR * 8, _z, 0)

    def _zs(t, carry):
        pltpu.sync_copy(rows_v, agg_sh.at[pl.ds(s * _RPS + t * _ZR, _ZR)])
        return carry
    lax.fori_loop(0, _NZ, _zs, 0)
    plsc.subcore_barrier()

    pltpu.sync_copy(srcb_hbm.at[wid], src_v)
    pltpu.sync_copy(dstb_hbm.at[wid], dst_v)

    def _edge(t, carry):
        pltpu.async_copy(h_hbm.at[src_v.at[t]], rows_v, sem).wait()
        pltpu.sync_copy(rows_v, agg_sh.at[dst_v.at[t]], add=True)
        return carry
    lax.fori_loop(0, _NB, _edge, 0)
    plsc.subcore_barrier()

    def _wo(t, carry):
        r0 = s * _RPS + t * _ZR
        pltpu.sync_copy(agg_sh.at[pl.ds(r0, _ZR)], rows_v)
        pltpu.sync_copy(rows_v, out_hbm.at[c, pl.ds(r0, _ZR)])
        return carry
    lax.fori_loop(0, _NZ, _wo, 0)


@functools.cache
def _sc_edge_agg():
    mesh = plsc.VectorSubcoreMesh(core_axis_name="c", subcore_axis_name="s",
                                  num_cores=_NC, num_subcores=_NS)
    return pl.kernel(
        _sc_edge_agg_body,
        out_type=jax.ShapeDtypeStruct((_NC, _NP, _D), jnp.float32),
        mesh=mesh,
        scratch_types=[
            pltpu.VMEM((_NB, _K), jnp.int32),
            pltpu.VMEM((_NB, _K), jnp.int32),
            pltpu.VMEM((_K, _D), jnp.float32),
            pltpu.VMEM_SHARED((_NP, _D), jnp.float32),
            pltpu.SemaphoreType.DMA,
        ],
    )


def _dot(a, b, dims):
    return lax.dot_general(a, b, (dims, ((), ())),
                           precision=lax.Precision.HIGHEST,
                           preferred_element_type=jnp.float32)


def _tc_layer_body(scale_ref, h_ref, p_ref, w_ref, b_ref, o_ref):
    z = scale_ref[0, 0] * h_ref[...] + p_ref[0] + p_ref[1]
    y = _dot(z, w_ref[...], ((1,), (0,)))
    o_ref[...] = jnp.maximum(y + b_ref[...], 0.0)


_tc_layer = pl.pallas_call(
    _tc_layer_body,
    grid=(_NBLK,),
    in_specs=[
        pl.BlockSpec(memory_space=pltpu.SMEM),
        pl.BlockSpec((_R, _D), lambda i: (i, 0)),
        pl.BlockSpec((_NC, _R, _D), lambda i: (0, i, 0)),
        pl.BlockSpec((_D, _D), lambda i: (0, 0)),
        pl.BlockSpec((1, _D), lambda i: (0, 0)),
    ],
    out_specs=pl.BlockSpec((_R, _D), lambda i: (i, 0)),
    out_shape=jax.ShapeDtypeStruct((_N, _D), jnp.float32),
)


def _tc_final_body(scale_ref, h_ref, p_ref, w4_ref, b4_ref, wn_ref, bn_ref,
                   wo_ref, bo_ref, bat_ref, o_ref, sums_ref, cnt_ref):
    i = pl.program_id(0)

    @pl.when(i == 0)
    def _():
        sums_ref[...] = jnp.zeros((_G, _D), jnp.float32)
        cnt_ref[...] = jnp.zeros((_G, _D), jnp.float32)

    z = scale_ref[0, 0] * h_ref[...] + p_ref[0] + p_ref[1]
    h5 = jnp.maximum(_dot(z, w4_ref[...], ((1,), (0,))) + b4_ref[...], 0.0)
    h6 = jnp.maximum(_dot(h5, wn_ref[...], ((1,), (0,))) + bn_ref[...], 0.0)
    gids = lax.broadcasted_iota(jnp.int32, (_R, _G), 1).astype(jnp.float32)
    oh = (bat_ref[...] == gids)
    oh = oh.astype(jnp.float32)
    sums_ref[...] += _dot(oh, h6, ((0,), (0,)))
    cnt_ref[...] += jnp.sum(oh, axis=0)[:, None]

    @pl.when(i == _NBLK - 1)
    def _():
        hg = sums_ref[...] / jnp.maximum(cnt_ref[...], 1.0)
        o_ref[...] = _dot(hg, wo_ref[...], ((1,), (0,))) + bo_ref[...]


_tc_final = pl.pallas_call(
    _tc_final_body,
    grid=(_NBLK,),
    in_specs=[
        pl.BlockSpec(memory_space=pltpu.SMEM),
        pl.BlockSpec((_R, _D), lambda i: (i, 0)),
        pl.BlockSpec((_NC, _R, _D), lambda i: (0, i, 0)),
        pl.BlockSpec((_D, _D), lambda i: (0, 0)),
        pl.BlockSpec((1, _D), lambda i: (0, 0)),
        pl.BlockSpec((_D, _D), lambda i: (0, 0)),
        pl.BlockSpec((1, _D), lambda i: (0, 0)),
        pl.BlockSpec((_D, _T), lambda i: (0, 0)),
        pl.BlockSpec((1, _T), lambda i: (0, 0)),
        pl.BlockSpec((_R, 1), lambda i: (i, 0)),
    ],
    out_specs=pl.BlockSpec((_G, _T), lambda i: (0, 0)),
    out_shape=jax.ShapeDtypeStruct((_G, _T), jnp.float32),
    scratch_shapes=[
        pltpu.VMEM((_G, _D), jnp.float32),
        pltpu.VMEM((_G, _D), jnp.float32),
    ],
)


def kernel(x, edge_index, batch, Wl, bl, eps, Wn2n, bn2n, Wout, bout):
    src_b = edge_index[0].reshape(_NW, _NB, _K)
    dst_b = edge_index[1].reshape(_NW, _NB, _K)
    batch_f = batch.astype(jnp.float32).reshape(_N, 1)
    h = x
    for l in range(_L):
        parts = _sc_edge_agg()(h, src_b, dst_b)
        scale = (1.0 + eps[l]).reshape(1, 1)
        b_l = bl[l].reshape(1, _D)
        if l < _L - 1:
            h = _tc_layer(scale, h, parts, Wl[l], b_l)
        else:
            out = _tc_final(scale, h, parts, Wl[l], b_l,
                            Wn2n, bn2n.reshape(1, _D),
                            Wout, bout.reshape(1, _T), batch_f)
    return out
